# baseline (speedup 1.0000x reference)
import functools

import jax
import jax.numpy as jnp
from jax import lax
from jax.experimental import pallas as pl
from jax.experimental.pallas import tpu as pltpu
from jax.experimental.pallas import tpu_sc as plsc

NUM_ROWS = 1000000
DIM = 32

B_TOTAL = 16384 * 200
G = 128
CHUNK = 1024
KJ = CHUNK // G
QROWS = G
PER_ROW = 128 // DIM
NW = 32
B_PER_W = B_TOTAL // NW
CHUNKS_PER_W = B_PER_W // CHUNK
NCHUNKS = B_TOTAL // CHUNK
OUT_ROWS = B_TOTAL * DIM // 128
CHUNK_OUT = CHUNK * DIM // 128


def _body(idx_hbm, w_hbm, out_hbm, idx_v, rows_v, sem):
    wid = lax.axis_index("s") * 2 + lax.axis_index("c")
    g0 = wid * CHUNKS_PER_W

    def chunk(i, carry):
        g = g0 + i
        pltpu.sync_copy(idx_hbm.at[g], idx_v)
        copies = [
            pltpu.async_copy(
                w_hbm.at[idx_v.at[j]],
                rows_v.at[pl.ds(j * G, G), :],
                sem,
            )
            for j in range(KJ)
        ]
        for c in copies:
            c.wait()
        wb = [
            pltpu.async_copy(
                rows_v.at[pl.ds(j * G, G), :],
                out_hbm.at[pl.ds(g * CHUNK_OUT + (j // PER_ROW) * G, G),
                           pl.ds((j % PER_ROW) * DIM, DIM)],
                sem,
            )
            for j in range(KJ)
        ]
        for c in wb:
            c.wait()
        return carry

    lax.fori_loop(0, CHUNKS_PER_W, chunk, 0)


_mesh = plsc.VectorSubcoreMesh(core_axis_name="c", subcore_axis_name="s")

_gather = functools.partial(
    pl.kernel,
    out_type=jax.ShapeDtypeStruct((OUT_ROWS, 128), jnp.float32),
    mesh=_mesh,
    scratch_types=[
        pltpu.VMEM((KJ, G), jnp.int32),
        pltpu.VMEM((CHUNK, DIM), jnp.float32),
        pltpu.SemaphoreType.DMA,
    ],
    compiler_params=pltpu.CompilerParams(use_tc_tiling_on_sc=False),
)(_body)


def kernel(x, weight):
    idx = x.reshape(NCHUNKS, CHUNK // (G * PER_ROW), G, PER_ROW)
    idx = idx.transpose(0, 1, 3, 2).reshape(NCHUNKS, KJ, G).astype(jnp.int32)
    out = _gather(idx, weight)
    return out.reshape(*x.shape, DIM)


# speedup vs baseline: 1.0294x; 1.0294x over previous
import functools

import jax
import jax.numpy as jnp
from jax import lax
from jax.experimental import pallas as pl
from jax.experimental.pallas import tpu as pltpu
from jax.experimental.pallas import tpu_sc as plsc

NUM_ROWS = 1000000
DIM = 32

B_TOTAL = 16384 * 200
G = 128
CHUNK = 1024
KJ = CHUNK // G
PER_ROW = 128 // DIM
NW = 32
B_PER_W = B_TOTAL // NW
CHUNKS_PER_W = B_PER_W // CHUNK
NPAIR = CHUNKS_PER_W // 2
NCHUNKS = B_TOTAL // CHUNK
OUT_ROWS = B_TOTAL * DIM // 128
CHUNK_OUT = CHUNK * DIM // 128


def _body(idx_hbm, w_hbm, out_hbm, idx_v, rows_v, sem_g, sem_w0, sem_w1):
    wid = lax.axis_index("s") * 2 + lax.axis_index("c")
    g0 = wid * CHUNKS_PER_W
    sem_w = (sem_w0, sem_w1)

    def issue_gathers(p):
        for j in range(KJ):
            pltpu.async_copy(
                w_hbm.at[idx_v.at[p, j]],
                rows_v.at[p, pl.ds(j * G, G), :],
                sem_g,
            )

    def drain_gathers(p):
        pltpu.make_async_copy(
            w_hbm.at[pl.ds(0, CHUNK), :], rows_v.at[p], sem_g
        ).wait()

    def issue_writebacks(g, p):
        for j in range(KJ):
            pltpu.async_copy(
                rows_v.at[p, pl.ds(j * G, G), :],
                out_hbm.at[pl.ds(g * CHUNK_OUT + (j // PER_ROW) * G, G),
                           pl.ds((j % PER_ROW) * DIM, DIM)],
                sem_w[p],
            )

    def drain_writebacks(p):
        pltpu.make_async_copy(
            w_hbm.at[pl.ds(0, CHUNK), :], rows_v.at[p], sem_w[p]
        ).wait()

    pltpu.sync_copy(idx_hbm.at[g0], idx_v.at[0])
    issue_gathers(0)

    def pair(cc, carry):
        c0 = g0 + 2 * cc

        drain_gathers(0)
        issue_writebacks(c0, 0)
        pltpu.sync_copy(idx_hbm.at[c0 + 1], idx_v.at[1])

        @pl.when(cc > 0)
        def _():
            drain_writebacks(1)
        issue_gathers(1)

        drain_gathers(1)
        issue_writebacks(c0 + 1, 1)

        @pl.when(cc < NPAIR - 1)
        def _():
            pltpu.sync_copy(idx_hbm.at[c0 + 2], idx_v.at[0])
            drain_writebacks(0)
            issue_gathers(0)
        return carry

    lax.fori_loop(0, NPAIR, pair, 0)
    drain_writebacks(0)
    drain_writebacks(1)


_mesh = plsc.VectorSubcoreMesh(core_axis_name="c", subcore_axis_name="s")

_gather = functools.partial(
    pl.kernel,
    out_type=jax.ShapeDtypeStruct((OUT_ROWS, 128), jnp.float32),
    mesh=_mesh,
    scratch_types=[
        pltpu.VMEM((2, KJ, G), jnp.int32),
        pltpu.VMEM((2, CHUNK, DIM), jnp.float32),
        pltpu.SemaphoreType.DMA,
        pltpu.SemaphoreType.DMA,
        pltpu.SemaphoreType.DMA,
    ],
    compiler_params=pltpu.CompilerParams(use_tc_tiling_on_sc=False),
)(_body)


def kernel(x, weight):
    idx = x.reshape(NCHUNKS, CHUNK // (G * PER_ROW), G, PER_ROW)
    idx = idx.transpose(0, 1, 3, 2).reshape(NCHUNKS, KJ, G).astype(jnp.int32)
    out = _gather(idx, weight)
    return out.reshape(*x.shape, DIM)


# speedup vs baseline: 1.0840x; 1.0531x over previous
import functools

import jax
import jax.numpy as jnp
from jax import lax
from jax.experimental import pallas as pl
from jax.experimental.pallas import tpu as pltpu
from jax.experimental.pallas import tpu_sc as plsc

NUM_ROWS = 1000000
DIM = 32

B_TOTAL = 16384 * 200
G = 128
CHUNK = 1024
KJ = CHUNK // G
NW = 32
B_PER_W = B_TOTAL // NW
CHUNKS_PER_W = B_PER_W // CHUNK
NPAIR = CHUNKS_PER_W // 2


def _body(idx_hbm, w_hbm, out_hbm, idx_v, rows_v, sem_g, sem_w0, sem_w1):
    wid = lax.axis_index("s") * 2 + lax.axis_index("c")
    base = wid * B_PER_W
    sem_w = (sem_w0, sem_w1)

    def stage_idx(c, p):
        pltpu.sync_copy(idx_hbm.at[pl.ds(base + c * CHUNK, CHUNK)],
                        idx_v.at[p])

    def issue_gathers(p):
        for j in range(KJ):
            pltpu.async_copy(
                w_hbm.at[idx_v.at[p, pl.ds(j * G, G)]],
                rows_v.at[p, pl.ds(j * G, G), :],
                sem_g,
            )

    def drain_gathers(p):
        pltpu.make_async_copy(
            w_hbm.at[pl.ds(0, CHUNK), :], rows_v.at[p], sem_g
        ).wait()

    def issue_writeback(c, p):
        pltpu.async_copy(
            rows_v.at[p],
            out_hbm.at[pl.ds(base + c * CHUNK, CHUNK), :],
            sem_w[p],
        )

    def drain_writeback(p):
        pltpu.make_async_copy(
            w_hbm.at[pl.ds(0, CHUNK), :], rows_v.at[p], sem_w[p]
        ).wait()

    stage_idx(0, 0)
    issue_gathers(0)

    def pair(cc, carry):
        c0 = 2 * cc

        drain_gathers(0)
        issue_writeback(c0, 0)
        stage_idx(c0 + 1, 1)

        @pl.when(cc > 0)
        def _():
            drain_writeback(1)
        issue_gathers(1)

        drain_gathers(1)
        issue_writeback(c0 + 1, 1)

        @pl.when(cc < NPAIR - 1)
        def _():
            stage_idx(c0 + 2, 0)
            drain_writeback(0)
            issue_gathers(0)
        return carry

    lax.fori_loop(0, NPAIR, pair, 0)
    drain_writeback(0)
    drain_writeback(1)


_mesh = plsc.VectorSubcoreMesh(core_axis_name="c", subcore_axis_name="s")

_gather = functools.partial(
    pl.kernel,
    out_type=jax.ShapeDtypeStruct((B_TOTAL, DIM), jnp.float32),
    mesh=_mesh,
    scratch_types=[
        pltpu.VMEM((2, CHUNK), jnp.int32),
        pltpu.VMEM((2, CHUNK, DIM), jnp.float32),
        pltpu.SemaphoreType.DMA,
        pltpu.SemaphoreType.DMA,
        pltpu.SemaphoreType.DMA,
    ],
    compiler_params=pltpu.CompilerParams(use_tc_tiling_on_sc=False),
)(_body)


def kernel(x, weight):
    idx = x.reshape(B_TOTAL).astype(jnp.int32)
    out = _gather(idx, weight)
    return out.reshape(*x.shape, DIM)


# speedup vs baseline: 1.8996x; 1.7523x over previous
import functools

import jax
import jax.numpy as jnp
from jax import lax
from jax.experimental import pallas as pl
from jax.experimental.pallas import tpu as pltpu
from jax.experimental.pallas import tpu_sc as plsc

NUM_ROWS = 1000000
DIM = 32

B_TOTAL = 16384 * 200
G = 128
CHUNK = 1024
KJ = CHUNK // G
NW = 32
B_PER_W = B_TOTAL // NW
CHUNKS_PER_W = B_PER_W // CHUNK
NPAIR = CHUNKS_PER_W // 2


def _body(idx_hbm, w_hbm, out_hbm, idx_v, rows_v, sem_g, sem_w0, sem_w1):
    wid = lax.axis_index("s") * 2 + lax.axis_index("c")
    base = wid * B_PER_W
    sem_w = (sem_w0, sem_w1)

    def stage_idx(c, p):
        pltpu.sync_copy(idx_hbm.at[pl.ds(base + c * CHUNK, CHUNK)],
                        idx_v.at[p])

    def issue_gathers(p):
        for j in range(KJ):
            pltpu.async_copy(
                w_hbm.at[idx_v.at[p, pl.ds(j * G, G)]],
                rows_v.at[p, pl.ds(j * G, G), :],
                sem_g,
            )

    def drain_gathers(p):
        pltpu.make_async_copy(
            w_hbm.at[pl.ds(0, CHUNK), :], rows_v.at[p], sem_g
        ).wait()

    def issue_writeback(c, p):
        pltpu.async_copy(
            rows_v.at[p],
            out_hbm.at[pl.ds(base + c * CHUNK, CHUNK), pl.ds(0, DIM)],
            sem_w[p],
        )

    def drain_writeback(p):
        pltpu.make_async_copy(
            w_hbm.at[pl.ds(0, CHUNK), :], rows_v.at[p], sem_w[p]
        ).wait()

    stage_idx(0, 0)
    issue_gathers(0)

    def pair(cc, carry):
        c0 = 2 * cc

        drain_gathers(0)
        issue_writeback(c0, 0)
        stage_idx(c0 + 1, 1)

        @pl.when(cc > 0)
        def _():
            drain_writeback(1)
        issue_gathers(1)

        drain_gathers(1)
        issue_writeback(c0 + 1, 1)

        @pl.when(cc < NPAIR - 1)
        def _():
            stage_idx(c0 + 2, 0)
            drain_writeback(0)
            issue_gathers(0)
        return carry

    lax.fori_loop(0, NPAIR, pair, 0)
    drain_writeback(0)
    drain_writeback(1)


_mesh = plsc.VectorSubcoreMesh(core_axis_name="c", subcore_axis_name="s")

_gather = functools.partial(
    pl.kernel,
    out_type=jax.ShapeDtypeStruct((B_TOTAL, 128), jnp.float32),
    mesh=_mesh,
    scratch_types=[
        pltpu.VMEM((2, CHUNK), jnp.int32),
        pltpu.VMEM((2, CHUNK, DIM), jnp.float32),
        pltpu.SemaphoreType.DMA,
        pltpu.SemaphoreType.DMA,
        pltpu.SemaphoreType.DMA,
    ],
    compiler_params=pltpu.CompilerParams(use_tc_tiling_on_sc=False),
)(_body)


def kernel(x, weight):
    idx = x.reshape(B_TOTAL).astype(jnp.int32)
    out = _gather(idx, weight)
    return out[:, :DIM].reshape(*x.shape, DIM)
